# Initial kernel scaffold; baseline (speedup 1.0000x reference)
import jax
import jax.numpy as jnp
from jax.experimental import pallas as pl


def kernel(x, Wr, br, Ws, gamma, beta, W1, W2):
    raise NotImplementedError("write your pallas kernel here")


# speedup vs baseline: 1.2545x; 1.2545x over previous
import jax
import jax.numpy as jnp
from jax.experimental import pallas as pl
from jax.experimental.pallas import tpu as pltpu

_B, _C, _H, _W = 64, 384, 14, 14
_E = 8
_K = 2
_HID = 2 * _C
_HW = _H * _W
_SB = 8


def _routing_kernel(x_ref, wr_ref, br_ref, idx_ref, wts_ref):
    xb = x_ref[...]
    pooled = jnp.mean(xb, axis=1)
    logits = jax.lax.dot_general(
        pooled, wr_ref[...], (((1,), (1,)), ((), ())),
        preferred_element_type=jnp.float32) + br_ref[...]
    probs = jax.nn.softmax(logits, axis=-1)
    lane = jax.lax.broadcasted_iota(jnp.int32, probs.shape, 1)
    a1 = jnp.argmax(probs, axis=-1)
    m1 = jnp.max(probs, axis=-1)
    masked = jnp.where(lane == a1[:, None], -jnp.inf, probs)
    a2 = jnp.argmax(masked, axis=-1)
    m2 = jnp.max(masked, axis=-1)
    denom = m1 + m2
    idx_ref[...] = jnp.concatenate([a1[:, None], a2[:, None]], axis=1)
    wts_ref[...] = jnp.concatenate(
        [(m1 / denom)[:, None], (m2 / denom)[:, None]], axis=1)


def _moe_kernel(idx_ref, wts_ref, x_ref, ws_ref, gamma_ref, beta_ref,
                w1_ref, w2_ref, out_ref):
    s = pl.program_id(0)
    k = pl.program_id(1)
    xb = x_ref[0]
    h = jnp.dot(xb, w1_ref[0], preferred_element_type=jnp.float32)
    h = h * jax.nn.sigmoid(h)
    eout = jnp.dot(h, w2_ref[0], preferred_element_type=jnp.float32)
    w = wts_ref[s, k]

    @pl.when(k == 0)
    def _():
        shared = jnp.dot(xb, ws_ref[...], preferred_element_type=jnp.float32)
        shared = shared * gamma_ref[...] + beta_ref[...]
        shared = shared * jax.nn.sigmoid(shared)
        out_ref[0] = xb + shared + w * eout

    @pl.when(k == 1)
    def _():
        out_ref[0] = out_ref[0] + w * eout


def kernel(x, Wr, br, Ws, gamma, beta, W1, W2):
    xt = x.reshape(_B, _C, _HW).transpose(0, 2, 1)

    idx, wts = pl.pallas_call(
        _routing_kernel,
        grid=(_B // _SB,),
        in_specs=[
            pl.BlockSpec((_SB, _HW, _C), lambda i: (i, 0, 0)),
            pl.BlockSpec((_E, _C), lambda i: (0, 0)),
            pl.BlockSpec((1, _E), lambda i: (0, 0)),
        ],
        out_specs=[
            pl.BlockSpec((_SB, _K), lambda i: (i, 0)),
            pl.BlockSpec((_SB, _K), lambda i: (i, 0)),
        ],
        out_shape=[
            jax.ShapeDtypeStruct((_B, _K), jnp.int32),
            jax.ShapeDtypeStruct((_B, _K), jnp.float32),
        ],
    )(xt, Wr, br.reshape(1, _E))

    ws_t = Ws.T
    w1_t = W1.transpose(0, 2, 1)
    w2_t = W2.transpose(0, 2, 1)

    grid_spec = pltpu.PrefetchScalarGridSpec(
        num_scalar_prefetch=2,
        grid=(_B, _K),
        in_specs=[
            pl.BlockSpec((1, _HW, _C), lambda s, k, idx, wts: (s, 0, 0)),
            pl.BlockSpec((_C, _C), lambda s, k, idx, wts: (0, 0)),
            pl.BlockSpec((1, _C), lambda s, k, idx, wts: (0, 0)),
            pl.BlockSpec((1, _C), lambda s, k, idx, wts: (0, 0)),
            pl.BlockSpec((1, _C, _HID), lambda s, k, idx, wts: (idx[s, k], 0, 0)),
            pl.BlockSpec((1, _HID, _C), lambda s, k, idx, wts: (idx[s, k], 0, 0)),
        ],
        out_specs=pl.BlockSpec((1, _HW, _C), lambda s, k, idx, wts: (s, 0, 0)),
    )
    out = pl.pallas_call(
        _moe_kernel,
        grid_spec=grid_spec,
        out_shape=jax.ShapeDtypeStruct((_B, _HW, _C), jnp.float32),
        compiler_params=pltpu.CompilerParams(
            dimension_semantics=("arbitrary", "arbitrary")),
    )(idx, wts, xt, ws_t, gamma.reshape(1, _C), beta.reshape(1, _C),
      w1_t, w2_t)

    return out.transpose(0, 2, 1).reshape(_B, _C, _H, _W)


# speedup vs baseline: 1.6919x; 1.3487x over previous
import jax
import jax.numpy as jnp
from jax.experimental import pallas as pl
from jax.experimental.pallas import tpu as pltpu

_B, _C, _H, _W = 64, 384, 14, 14
_E = 8
_K = 2
_HID = 2 * _C
_HW = _H * _W
_SB = 8


def _routing_kernel(x_ref, wr_ref, br_ref, idx_ref, wts_ref):
    xb = x_ref[...]
    pooled = jnp.mean(xb, axis=2)
    logits = jax.lax.dot_general(
        pooled, wr_ref[...], (((1,), (1,)), ((), ())),
        preferred_element_type=jnp.float32) + br_ref[...]
    probs = jax.nn.softmax(logits, axis=-1)
    lane = jax.lax.broadcasted_iota(jnp.int32, probs.shape, 1)
    a1 = jnp.argmax(probs, axis=-1)
    m1 = jnp.max(probs, axis=-1)
    masked = jnp.where(lane == a1[:, None], -jnp.inf, probs)
    a2 = jnp.argmax(masked, axis=-1)
    m2 = jnp.max(masked, axis=-1)
    denom = m1 + m2
    idx_ref[...] = jnp.concatenate([a1[:, None], a2[:, None]], axis=1)
    wts_ref[...] = jnp.concatenate(
        [(m1 / denom)[:, None], (m2 / denom)[:, None]], axis=1)


def _moe_kernel(idx_ref, wts_ref, x_ref, ws_ref, gamma_ref, beta_ref,
                w1_ref, w2_ref, out_ref):
    s = pl.program_id(0)
    xb = x_ref[0]
    e0 = idx_ref[s, 0]
    e1 = idx_ref[s, 1]
    w0 = wts_ref[s, 0]
    w1 = wts_ref[s, 1]

    h0 = jnp.dot(w1_ref[e0], xb, preferred_element_type=jnp.float32)
    h0 = h0 * jax.nn.sigmoid(h0)
    out0 = jnp.dot(w2_ref[e0], h0, preferred_element_type=jnp.float32)

    h1 = jnp.dot(w1_ref[e1], xb, preferred_element_type=jnp.float32)
    h1 = h1 * jax.nn.sigmoid(h1)
    out1 = jnp.dot(w2_ref[e1], h1, preferred_element_type=jnp.float32)

    shared = jnp.dot(ws_ref[...], xb, preferred_element_type=jnp.float32)
    shared = shared * gamma_ref[...] + beta_ref[...]
    shared = shared * jax.nn.sigmoid(shared)

    out_ref[0] = xb + shared + w0 * out0 + w1 * out1


def kernel(x, Wr, br, Ws, gamma, beta, W1, W2):
    xr = x.reshape(_B, _C, _HW)

    idx, wts = pl.pallas_call(
        _routing_kernel,
        grid=(_B // _SB,),
        in_specs=[
            pl.BlockSpec((_SB, _C, _HW), lambda i: (i, 0, 0)),
            pl.BlockSpec((_E, _C), lambda i: (0, 0)),
            pl.BlockSpec((1, _E), lambda i: (0, 0)),
        ],
        out_specs=[
            pl.BlockSpec((_SB, _K), lambda i: (i, 0)),
            pl.BlockSpec((_SB, _K), lambda i: (i, 0)),
        ],
        out_shape=[
            jax.ShapeDtypeStruct((_B, _K), jnp.int32),
            jax.ShapeDtypeStruct((_B, _K), jnp.float32),
        ],
    )(xr, Wr, br.reshape(1, _E))

    grid_spec = pltpu.PrefetchScalarGridSpec(
        num_scalar_prefetch=2,
        grid=(_B,),
        in_specs=[
            pl.BlockSpec((1, _C, _HW), lambda s, idx, wts: (s, 0, 0)),
            pl.BlockSpec((_C, _C), lambda s, idx, wts: (0, 0)),
            pl.BlockSpec((_C, 1), lambda s, idx, wts: (0, 0)),
            pl.BlockSpec((_C, 1), lambda s, idx, wts: (0, 0)),
            pl.BlockSpec((_E, _HID, _C), lambda s, idx, wts: (0, 0, 0)),
            pl.BlockSpec((_E, _C, _HID), lambda s, idx, wts: (0, 0, 0)),
        ],
        out_specs=pl.BlockSpec((1, _C, _HW), lambda s, idx, wts: (s, 0, 0)),
    )
    out = pl.pallas_call(
        _moe_kernel,
        grid_spec=grid_spec,
        out_shape=jax.ShapeDtypeStruct((_B, _C, _HW), jnp.float32),
        compiler_params=pltpu.CompilerParams(
            dimension_semantics=("arbitrary",)),
    )(idx, wts, xr, Ws, gamma.reshape(_C, 1), beta.reshape(_C, 1), W1, W2)

    return out.reshape(_B, _C, _H, _W)
